# Initial kernel scaffold; baseline (speedup 1.0000x reference)
import jax
import jax.numpy as jnp
from jax import lax
from jax.experimental import pallas as pl

BETA = 0.25

_PRECISION = lax.Precision.HIGHEST


def kernel(z, codebook):
    b, c, h, w = z.shape
    z_perm = jnp.transpose(z, (0, 2, 3, 1))
    z_flat = z_perm.reshape(-1, c)
    z_sq = jnp.sum(z_flat ** 2, axis=1, keepdims=True)
    e_sq = jnp.sum(codebook ** 2, axis=1)
    ze = jnp.dot(z_flat, codebook.T, precision=_PRECISION)
    dists = z_sq + e_sq[None, :] - 2.0 * ze
    indices = jnp.argmin(dists, axis=1)
    z_q = jnp.take(codebook, indices, axis=0).reshape(b, h, w, c)
    z_q = jnp.transpose(z_q, (0, 3, 1, 2))
    z_q_st = z + jax.lax.stop_gradient(z_q - z)
    commitment = jnp.mean((jax.lax.stop_gradient(z) - z_q) ** 2)
    codebook_loss = jnp.mean((z - jax.lax.stop_gradient(z_q)) ** 2)
    vq_loss = codebook_loss + BETA * commitment
    return (z_q_st, vq_loss, indices.reshape(b, h, w))


# speedup vs baseline: 1.0392x; 1.0392x over previous
import jax
import jax.numpy as jnp
from jax import lax
from jax.experimental import pallas as pl
from jax.experimental.pallas import tpu as pltpu
from jax.experimental.pallas import tpu_sc as plsc

BETA = 0.25

B, C, H, W = 16, 256, 32, 32
HW = H * W
N = B * HW
NCODES = 8192
CT = 512
NT = NCODES // CT
ROWS = CT // 8

G1 = 2736 // 8
G2 = 5472 // 8


def _dist_argmin_body(cb2_ref, esq_ref, z_ref, zsq_ref,
                      idx_ref, loss_ref, val_st, row_st, acc_st):
    b = pl.program_id(0)
    t = pl.program_id(1)

    @pl.when(t == 0)
    def _init():
        val_st[...] = jnp.full((24, HW), jnp.inf, jnp.float32)
        row_st[...] = jnp.zeros((24, HW), jnp.int32)

    @pl.when((b == 0) & (t == 0))
    def _init_acc():
        acc_st[0] = 0.0

    cb_tile = cb2_ref[pl.ds(t * CT, CT), :]
    esq_tile = esq_ref[pl.ds(t * CT, CT), :]
    zb = z_ref[0]
    zsq = zsq_ref[0]

    ze2 = lax.dot_general(cb_tile, zb, (((1,), (0,)), ((), ())),
                          precision=lax.Precision.DEFAULT,
                          preferred_element_type=jnp.float32)
    d = (esq_tile + zsq) - ze2

    def upd(chunk, r0, r1):
        val = val_st[pl.ds(chunk * 8, 8), :]
        row = row_st[pl.ds(chunk * 8, 8), :]
        for r in range(r0, r1):
            dr = lax.slice(d, (r * 8, 0), (r * 8 + 8, HW))
            m = dr < val
            val = jnp.where(m, dr, val)
            row = jnp.where(m, t * ROWS + r, row)
        val_st[pl.ds(chunk * 8, 8), :] = val
        row_st[pl.ds(chunk * 8, 8), :] = row

    @pl.when(t < 5)
    def _c0():
        upd(0, 0, ROWS)

    @pl.when(t == 5)
    def _c01():
        upd(0, 0, G1 - 5 * ROWS)
        upd(1, G1 - 5 * ROWS, ROWS)

    @pl.when((t > 5) & (t < 10))
    def _c1():
        upd(1, 0, ROWS)

    @pl.when(t == 10)
    def _c12():
        upd(1, 0, G2 - 10 * ROWS)
        upd(2, G2 - 10 * ROWS, ROWS)

    @pl.when(t > 10)
    def _c2():
        upd(2, 0, ROWS)

    @pl.when(t == NT - 1)
    def _finalize():
        mvs, mjs = [], []
        for chunk in range(3):
            v = val_st[pl.ds(chunk * 8, 8), :]
            j = (row_st[pl.ds(chunk * 8, 8), :] * 8
                 + lax.broadcasted_iota(jnp.int32, (8, HW), 0))
            hv, hj = v, j
            for half in (4, 2, 1):
                v2 = lax.slice(hv, (half, 0), (2 * half, HW))
                j2 = lax.slice(hj, (half, 0), (2 * half, HW))
                v1 = lax.slice(hv, (0, 0), (half, HW))
                j1 = lax.slice(hj, (0, 0), (half, HW))
                take = (v2 < v1) | ((v2 == v1) & (j2 < j1))
                hv = jnp.where(take, v2, v1)
                hj = jnp.where(take, j2, j1)
            mvs.append(hv)
            mjs.append(hj)

        accv = mvs[0].astype(jnp.bfloat16).astype(jnp.float32)
        acci = mjs[0]
        selv = mvs[0]
        for chunk in (1, 2):
            take = mvs[chunk] < accv
            selv = jnp.where(take, mvs[chunk], selv)
            acci = jnp.where(take, mjs[chunk], acci)
            accv = jnp.where(take, mvs[chunk], accv)
            accv = accv.astype(jnp.bfloat16).astype(jnp.float32)

        idx_ref[0] = acci
        acc_st[0] += jnp.sum(selv)

        @pl.when(b == B - 1)
        def _write_loss():
            loss_ref[...] = jnp.full((1, 1), acc_st[0], jnp.float32)


def _dist_argmin(cb2, esq, z3, zsq3):
    return pl.pallas_call(
        _dist_argmin_body,
        grid=(B, NT),
        in_specs=[
            pl.BlockSpec((NCODES, C), lambda b, t: (0, 0)),
            pl.BlockSpec((NCODES, 1), lambda b, t: (0, 0)),
            pl.BlockSpec((1, C, HW), lambda b, t: (b, 0, 0)),
            pl.BlockSpec((1, 1, HW), lambda b, t: (b, 0, 0)),
        ],
        out_specs=[
            pl.BlockSpec((1, 1, HW), lambda b, t: (b, 0, 0)),
            pl.BlockSpec((1, 1), lambda b, t: (0, 0)),
        ],
        out_shape=[
            jax.ShapeDtypeStruct((B, 1, HW), jnp.int32),
            jax.ShapeDtypeStruct((1, 1), jnp.float32),
        ],
        scratch_shapes=[
            pltpu.VMEM((24, HW), jnp.float32),
            pltpu.VMEM((24, HW), jnp.int32),
            pltpu.SMEM((1,), jnp.float32),
        ],
    )(cb2, esq, z3, zsq3)


def _sc_gather(codebook, idx_flat):
    window = 128
    mesh = plsc.VectorSubcoreMesh(core_axis_name="core",
                                  subcore_axis_name="subcore")

    @pl.kernel(out_type=jax.ShapeDtypeStruct((N, C), jnp.float32), mesh=mesh)
    def gather_kernel(cb_hbm, i_hbm, o_hbm):
        def body(i_vmem, o_vmem):
            pltpu.sync_copy(cb_hbm.at[i_vmem.at[0]], o_vmem)

        pltpu.emit_pipeline(
            body,
            grid=(N // window,),
            in_specs=[pl.BlockSpec((1, window), index_map=lambda i: (0, i))],
            out_specs=[pl.BlockSpec((window, C), index_map=lambda i: (i, 0))],
            core_axis_name=("core", "subcore"),
            dimension_semantics=(pltpu.PARALLEL,),
        )(i_hbm, o_hbm)

    return gather_kernel(codebook, idx_flat.reshape(1, N))


def kernel(z, codebook):
    b, c, h, w = z.shape
    z_perm = jnp.transpose(z, (0, 2, 3, 1))
    z_flat = z_perm.reshape(-1, c)
    z_sq = jnp.sum(z_flat ** 2, axis=1, keepdims=True)
    e_sq = jnp.sum(codebook ** 2, axis=1)

    cb2 = codebook * 2.0
    z3 = z.reshape(b, c, h * w)
    zsq3 = z_sq.reshape(b, 1, h * w)
    esq2 = e_sq.reshape(NCODES, 1)

    idx3, loss_sum = _dist_argmin(cb2, esq2, z3, zsq3)
    indices = idx3.reshape(b, h, w)

    z_q_rows = _sc_gather(codebook, idx3)
    z_q = z_q_rows.reshape(b, h * w, c).transpose(0, 2, 1).reshape(b, c, h, w)

    z_q_st = z + (z_q - z)
    vq_loss = (loss_sum[0, 0] / jnp.float32(z.size)) * jnp.float32(1.0 + BETA)
    return (z_q_st, vq_loss, indices)


# speedup vs baseline: 1.1468x; 1.1035x over previous
import jax
import jax.numpy as jnp
from jax import lax
from jax.experimental import pallas as pl
from jax.experimental.pallas import tpu as pltpu
from jax.experimental.pallas import tpu_sc as plsc

BETA = 0.25

B, C, H, W = 16, 256, 32, 32
HW = H * W
N = B * HW
NCODES = 8192
CT = 1024
NT = NCODES // CT
ROWS = CT // 8

G1 = 2736 // 8
G2 = 5472 // 8
T1, R1 = divmod(G1, ROWS)
T2, R2 = divmod(G2, ROWS)


def _dist_argmin_body(cb2_ref, esq_ref, z_ref, zsq_ref,
                      idx_ref, loss_ref, val_st, row_st, acc_st):
    b = pl.program_id(0)
    t = pl.program_id(1)

    @pl.when(t == 0)
    def _init():
        val_st[...] = jnp.full((24, HW), jnp.inf, jnp.float32)
        row_st[...] = jnp.zeros((24, HW), jnp.int32)

    @pl.when((b == 0) & (t == 0))
    def _init_acc():
        acc_st[0] = 0.0

    cb_tile = cb2_ref[pl.ds(t * CT, CT), :]
    esq_tile = esq_ref[pl.ds(t * CT, CT), :]
    zb = z_ref[0]
    zsq = zsq_ref[0]

    ze2 = lax.dot_general(cb_tile, zb, (((1,), (0,)), ((), ())),
                          precision=lax.Precision.DEFAULT,
                          preferred_element_type=jnp.float32)
    d = (esq_tile + zsq) - ze2

    def upd(chunk, r0, r1):
        val = val_st[pl.ds(chunk * 8, 8), :]
        row = row_st[pl.ds(chunk * 8, 8), :]
        for r in range(r0, r1):
            dr = lax.slice(d, (r * 8, 0), (r * 8 + 8, HW))
            m = dr < val
            val = jnp.where(m, dr, val)
            row = jnp.where(m, t * ROWS + r, row)
        val_st[pl.ds(chunk * 8, 8), :] = val
        row_st[pl.ds(chunk * 8, 8), :] = row

    @pl.when(t < T1)
    def _c0():
        upd(0, 0, ROWS)

    @pl.when(t == T1)
    def _c01():
        upd(0, 0, R1)
        upd(1, R1, ROWS)

    @pl.when((t > T1) & (t < T2))
    def _c1():
        upd(1, 0, ROWS)

    @pl.when(t == T2)
    def _c12():
        upd(1, 0, R2)
        upd(2, R2, ROWS)

    @pl.when(t > T2)
    def _c2():
        upd(2, 0, ROWS)

    @pl.when(t == NT - 1)
    def _finalize():
        mvs, mjs = [], []
        for chunk in range(3):
            v = val_st[pl.ds(chunk * 8, 8), :]
            j = (row_st[pl.ds(chunk * 8, 8), :] * 8
                 + lax.broadcasted_iota(jnp.int32, (8, HW), 0))
            hv, hj = v, j
            for half in (4, 2, 1):
                v2 = lax.slice(hv, (half, 0), (2 * half, HW))
                j2 = lax.slice(hj, (half, 0), (2 * half, HW))
                v1 = lax.slice(hv, (0, 0), (half, HW))
                j1 = lax.slice(hj, (0, 0), (half, HW))
                take = (v2 < v1) | ((v2 == v1) & (j2 < j1))
                hv = jnp.where(take, v2, v1)
                hj = jnp.where(take, j2, j1)
            mvs.append(hv)
            mjs.append(hj)

        accv = mvs[0].astype(jnp.bfloat16).astype(jnp.float32)
        acci = mjs[0]
        selv = mvs[0]
        for chunk in (1, 2):
            take = mvs[chunk] < accv
            selv = jnp.where(take, mvs[chunk], selv)
            acci = jnp.where(take, mjs[chunk], acci)
            accv = jnp.where(take, mvs[chunk], accv)
            accv = accv.astype(jnp.bfloat16).astype(jnp.float32)

        idx_ref[0] = acci
        acc_st[0] += jnp.sum(selv)

        @pl.when(b == B - 1)
        def _write_loss():
            loss_ref[...] = jnp.full((1, 1), acc_st[0], jnp.float32)


def _dist_argmin(cb2, esq, z3, zsq3):
    return pl.pallas_call(
        _dist_argmin_body,
        grid=(B, NT),
        in_specs=[
            pl.BlockSpec((NCODES, C), lambda b, t: (0, 0)),
            pl.BlockSpec((NCODES, 1), lambda b, t: (0, 0)),
            pl.BlockSpec((1, C, HW), lambda b, t: (b, 0, 0)),
            pl.BlockSpec((1, 1, HW), lambda b, t: (b, 0, 0)),
        ],
        out_specs=[
            pl.BlockSpec((1, 1, HW), lambda b, t: (b, 0, 0)),
            pl.BlockSpec((1, 1), lambda b, t: (0, 0)),
        ],
        out_shape=[
            jax.ShapeDtypeStruct((B, 1, HW), jnp.int32),
            jax.ShapeDtypeStruct((1, 1), jnp.float32),
        ],
        scratch_shapes=[
            pltpu.VMEM((24, HW), jnp.float32),
            pltpu.VMEM((24, HW), jnp.int32),
            pltpu.SMEM((1,), jnp.float32),
        ],
    )(cb2, esq, z3, zsq3)


def _sc_gather(codebook, idx_flat):
    window = 128
    mesh = plsc.VectorSubcoreMesh(core_axis_name="core",
                                  subcore_axis_name="subcore")

    @pl.kernel(out_type=jax.ShapeDtypeStruct((N, C), jnp.float32), mesh=mesh)
    def gather_kernel(cb_hbm, i_hbm, o_hbm):
        def body(i_vmem, o_vmem):
            pltpu.sync_copy(cb_hbm.at[i_vmem.at[0]], o_vmem)

        pltpu.emit_pipeline(
            body,
            grid=(N // window,),
            in_specs=[pl.BlockSpec((1, window), index_map=lambda i: (0, i))],
            out_specs=[pl.BlockSpec((window, C), index_map=lambda i: (i, 0))],
            core_axis_name=("core", "subcore"),
            dimension_semantics=(pltpu.PARALLEL,),
        )(i_hbm, o_hbm)

    return gather_kernel(codebook, idx_flat.reshape(1, N))


def kernel(z, codebook):
    b, c, h, w = z.shape
    z_perm = jnp.transpose(z, (0, 2, 3, 1))
    z_flat = z_perm.reshape(-1, c)
    z_sq = jnp.sum(z_flat ** 2, axis=1, keepdims=True)
    e_sq = jnp.sum(codebook ** 2, axis=1)

    cb2 = codebook * 2.0
    z3 = z.reshape(b, c, h * w)
    zsq3 = z_sq.reshape(b, 1, h * w)
    esq2 = e_sq.reshape(NCODES, 1)

    idx3, loss_sum = _dist_argmin(cb2, esq2, z3, zsq3)
    indices = idx3.reshape(b, h, w)

    z_q_rows = _sc_gather(codebook, idx3)
    z_q = z_q_rows.reshape(b, h * w, c).transpose(0, 2, 1).reshape(b, c, h, w)

    z_q_st = z + (z_q - z)
    vq_loss = (loss_sum[0, 0] / jnp.float32(z.size)) * jnp.float32(1.0 + BETA)
    return (z_q_st, vq_loss, indices)


# speedup vs baseline: 1.2036x; 1.0496x over previous
import jax
import jax.numpy as jnp
from jax import lax
from jax.experimental import pallas as pl
from jax.experimental.pallas import tpu as pltpu
from jax.experimental.pallas import tpu_sc as plsc

BETA = 0.25

B, C, H, W = 16, 256, 32, 32
HW = H * W
N = B * HW
NCODES = 8192
CT = 2048
NT = NCODES // CT
ROWS = CT // 8

G1 = 2736 // 8
G2 = 5472 // 8
T1, R1 = divmod(G1, ROWS)
T2, R2 = divmod(G2, ROWS)


def _dist_argmin_body(cb2_ref, esq_ref, z_ref, zsq_ref,
                      idx_ref, loss_ref, val_st, row_st, acc_st):
    b = pl.program_id(0)
    t = pl.program_id(1)

    @pl.when(t == 0)
    def _init():
        val_st[...] = jnp.full((24, HW), jnp.inf, jnp.float32)
        row_st[...] = jnp.zeros((24, HW), jnp.int32)

    @pl.when((b == 0) & (t == 0))
    def _init_acc():
        acc_st[0] = 0.0

    cb_tile = cb2_ref[pl.ds(t * CT, CT), :]
    esq_tile = esq_ref[pl.ds(t * CT, CT), :]
    zb = z_ref[0]
    zsq = zsq_ref[0]

    ze2 = lax.dot_general(cb_tile, zb, (((1,), (0,)), ((), ())),
                          precision=lax.Precision.DEFAULT,
                          preferred_element_type=jnp.float32)
    d = (esq_tile + zsq) - ze2

    def upd(chunk, r0, r1):
        val = val_st[pl.ds(chunk * 8, 8), :]
        row = row_st[pl.ds(chunk * 8, 8), :]
        for r in range(r0, r1):
            dr = lax.slice(d, (r * 8, 0), (r * 8 + 8, HW))
            m = dr < val
            val = jnp.where(m, dr, val)
            row = jnp.where(m, t * ROWS + r, row)
        val_st[pl.ds(chunk * 8, 8), :] = val
        row_st[pl.ds(chunk * 8, 8), :] = row

    @pl.when(t < T1)
    def _c0():
        upd(0, 0, ROWS)

    @pl.when(t == T1)
    def _c01():
        upd(0, 0, R1)
        upd(1, R1, ROWS)

    @pl.when((t > T1) & (t < T2))
    def _c1():
        upd(1, 0, ROWS)

    @pl.when(t == T2)
    def _c12():
        upd(1, 0, R2)
        upd(2, R2, ROWS)

    @pl.when(t > T2)
    def _c2():
        upd(2, 0, ROWS)

    @pl.when(t == NT - 1)
    def _finalize():
        mvs, mjs = [], []
        for chunk in range(3):
            v = val_st[pl.ds(chunk * 8, 8), :]
            j = (row_st[pl.ds(chunk * 8, 8), :] * 8
                 + lax.broadcasted_iota(jnp.int32, (8, HW), 0))
            hv, hj = v, j
            for half in (4, 2, 1):
                v2 = lax.slice(hv, (half, 0), (2 * half, HW))
                j2 = lax.slice(hj, (half, 0), (2 * half, HW))
                v1 = lax.slice(hv, (0, 0), (half, HW))
                j1 = lax.slice(hj, (0, 0), (half, HW))
                take = (v2 < v1) | ((v2 == v1) & (j2 < j1))
                hv = jnp.where(take, v2, v1)
                hj = jnp.where(take, j2, j1)
            mvs.append(hv)
            mjs.append(hj)

        accv = mvs[0].astype(jnp.bfloat16).astype(jnp.float32)
        acci = mjs[0]
        selv = mvs[0]
        for chunk in (1, 2):
            take = mvs[chunk] < accv
            selv = jnp.where(take, mvs[chunk], selv)
            acci = jnp.where(take, mjs[chunk], acci)
            accv = jnp.where(take, mvs[chunk], accv)
            accv = accv.astype(jnp.bfloat16).astype(jnp.float32)

        idx_ref[0] = acci
        acc_st[0] += jnp.sum(selv)

        @pl.when(b == B - 1)
        def _write_loss():
            loss_ref[...] = jnp.full((1, 1), acc_st[0], jnp.float32)


def _dist_argmin(cb2, esq, z3, zsq3):
    return pl.pallas_call(
        _dist_argmin_body,
        grid=(B, NT),
        in_specs=[
            pl.BlockSpec((NCODES, C), lambda b, t: (0, 0)),
            pl.BlockSpec((NCODES, 1), lambda b, t: (0, 0)),
            pl.BlockSpec((1, C, HW), lambda b, t: (b, 0, 0)),
            pl.BlockSpec((1, 1, HW), lambda b, t: (b, 0, 0)),
        ],
        out_specs=[
            pl.BlockSpec((1, 1, HW), lambda b, t: (b, 0, 0)),
            pl.BlockSpec((1, 1), lambda b, t: (0, 0)),
        ],
        out_shape=[
            jax.ShapeDtypeStruct((B, 1, HW), jnp.int32),
            jax.ShapeDtypeStruct((1, 1), jnp.float32),
        ],
        scratch_shapes=[
            pltpu.VMEM((24, HW), jnp.float32),
            pltpu.VMEM((24, HW), jnp.int32),
            pltpu.SMEM((1,), jnp.float32),
        ],
    )(cb2, esq, z3, zsq3)


def _sc_gather(codebook, idx_flat):
    window = 128
    mesh = plsc.VectorSubcoreMesh(core_axis_name="core",
                                  subcore_axis_name="subcore")

    @pl.kernel(out_type=jax.ShapeDtypeStruct((N, C), jnp.float32), mesh=mesh)
    def gather_kernel(cb_hbm, i_hbm, o_hbm):
        def body(i_vmem, o_vmem):
            pltpu.sync_copy(cb_hbm.at[i_vmem.at[0]], o_vmem)

        pltpu.emit_pipeline(
            body,
            grid=(N // window,),
            in_specs=[pl.BlockSpec((1, window), index_map=lambda i: (0, i))],
            out_specs=[pl.BlockSpec((window, C), index_map=lambda i: (i, 0))],
            core_axis_name=("core", "subcore"),
            dimension_semantics=(pltpu.PARALLEL,),
        )(i_hbm, o_hbm)

    return gather_kernel(codebook, idx_flat.reshape(1, N))


def kernel(z, codebook):
    b, c, h, w = z.shape
    z_perm = jnp.transpose(z, (0, 2, 3, 1))
    z_flat = z_perm.reshape(-1, c)
    z_sq = jnp.sum(z_flat ** 2, axis=1, keepdims=True)
    e_sq = jnp.sum(codebook ** 2, axis=1)

    cb2 = codebook * 2.0
    z3 = z.reshape(b, c, h * w)
    zsq3 = z_sq.reshape(b, 1, h * w)
    esq2 = e_sq.reshape(NCODES, 1)

    idx3, loss_sum = _dist_argmin(cb2, esq2, z3, zsq3)
    indices = idx3.reshape(b, h, w)

    z_q_rows = _sc_gather(codebook, idx3)
    z_q = z_q_rows.reshape(b, h * w, c).transpose(0, 2, 1).reshape(b, c, h, w)

    z_q_st = z + (z_q - z)
    vq_loss = (loss_sum[0, 0] / jnp.float32(z.size)) * jnp.float32(1.0 + BETA)
    return (z_q_st, vq_loss, indices)


# speedup vs baseline: 1.2390x; 1.0294x over previous
import jax
import jax.numpy as jnp
from jax import lax
from jax.experimental import pallas as pl
from jax.experimental.pallas import tpu as pltpu
from jax.experimental.pallas import tpu_sc as plsc

BETA = 0.25

B, C, H, W = 16, 256, 32, 32
HW = H * W
N = B * HW
NCODES = 8192
CT = 4096
NT = NCODES // CT
ROWS = CT // 8

G1 = 2736 // 8
G2 = 5472 // 8
T1, R1 = divmod(G1, ROWS)
T2, R2 = divmod(G2, ROWS)


def _dist_argmin_body(cb2_ref, esq_ref, z_ref, zsq_ref,
                      idx_ref, loss_ref, val_st, row_st, acc_st):
    b = pl.program_id(0)
    t = pl.program_id(1)

    @pl.when(t == 0)
    def _init():
        val_st[...] = jnp.full((24, HW), jnp.inf, jnp.float32)
        row_st[...] = jnp.zeros((24, HW), jnp.int32)

    @pl.when((b == 0) & (t == 0))
    def _init_acc():
        acc_st[0] = 0.0

    cb_tile = cb2_ref[pl.ds(t * CT, CT), :]
    esq_tile = esq_ref[pl.ds(t * CT, CT), :]
    zb = z_ref[0]
    zsq = zsq_ref[0]

    ze2 = lax.dot_general(cb_tile, zb, (((1,), (0,)), ((), ())),
                          precision=lax.Precision.DEFAULT,
                          preferred_element_type=jnp.float32)
    d = (esq_tile + zsq) - ze2

    def upd(chunk, r0, r1):
        val = val_st[pl.ds(chunk * 8, 8), :]
        row = row_st[pl.ds(chunk * 8, 8), :]
        for r in range(r0, r1):
            dr = lax.slice(d, (r * 8, 0), (r * 8 + 8, HW))
            m = dr < val
            val = jnp.where(m, dr, val)
            row = jnp.where(m, t * ROWS + r, row)
        val_st[pl.ds(chunk * 8, 8), :] = val
        row_st[pl.ds(chunk * 8, 8), :] = row

    @pl.when(t < T1)
    def _c0():
        upd(0, 0, ROWS)

    @pl.when(t == T1)
    def _c01():
        upd(0, 0, R1)
        upd(1, R1, ROWS)

    @pl.when((t > T1) & (t < T2))
    def _c1():
        upd(1, 0, ROWS)

    @pl.when(t == T2)
    def _c12():
        upd(1, 0, R2)
        upd(2, R2, ROWS)

    @pl.when(t > T2)
    def _c2():
        upd(2, 0, ROWS)

    @pl.when(t == NT - 1)
    def _finalize():
        mvs, mjs = [], []
        for chunk in range(3):
            v = val_st[pl.ds(chunk * 8, 8), :]
            j = (row_st[pl.ds(chunk * 8, 8), :] * 8
                 + lax.broadcasted_iota(jnp.int32, (8, HW), 0))
            hv, hj = v, j
            for half in (4, 2, 1):
                v2 = lax.slice(hv, (half, 0), (2 * half, HW))
                j2 = lax.slice(hj, (half, 0), (2 * half, HW))
                v1 = lax.slice(hv, (0, 0), (half, HW))
                j1 = lax.slice(hj, (0, 0), (half, HW))
                take = (v2 < v1) | ((v2 == v1) & (j2 < j1))
                hv = jnp.where(take, v2, v1)
                hj = jnp.where(take, j2, j1)
            mvs.append(hv)
            mjs.append(hj)

        accv = mvs[0].astype(jnp.bfloat16).astype(jnp.float32)
        acci = mjs[0]
        selv = mvs[0]
        for chunk in (1, 2):
            take = mvs[chunk] < accv
            selv = jnp.where(take, mvs[chunk], selv)
            acci = jnp.where(take, mjs[chunk], acci)
            accv = jnp.where(take, mvs[chunk], accv)
            accv = accv.astype(jnp.bfloat16).astype(jnp.float32)

        idx_ref[0] = acci
        acc_st[0] += jnp.sum(selv)

        @pl.when(b == B - 1)
        def _write_loss():
            loss_ref[...] = jnp.full((1, 1), acc_st[0], jnp.float32)


def _dist_argmin(cb2, esq, z3, zsq3):
    return pl.pallas_call(
        _dist_argmin_body,
        grid=(B, NT),
        in_specs=[
            pl.BlockSpec((NCODES, C), lambda b, t: (0, 0)),
            pl.BlockSpec((NCODES, 1), lambda b, t: (0, 0)),
            pl.BlockSpec((1, C, HW), lambda b, t: (b, 0, 0)),
            pl.BlockSpec((1, 1, HW), lambda b, t: (b, 0, 0)),
        ],
        out_specs=[
            pl.BlockSpec((1, 1, HW), lambda b, t: (b, 0, 0)),
            pl.BlockSpec((1, 1), lambda b, t: (0, 0)),
        ],
        out_shape=[
            jax.ShapeDtypeStruct((B, 1, HW), jnp.int32),
            jax.ShapeDtypeStruct((1, 1), jnp.float32),
        ],
        scratch_shapes=[
            pltpu.VMEM((24, HW), jnp.float32),
            pltpu.VMEM((24, HW), jnp.int32),
            pltpu.SMEM((1,), jnp.float32),
        ],
    )(cb2, esq, z3, zsq3)


def _sc_gather(codebook, idx_flat):
    window = 128
    mesh = plsc.VectorSubcoreMesh(core_axis_name="core",
                                  subcore_axis_name="subcore")

    @pl.kernel(out_type=jax.ShapeDtypeStruct((N, C), jnp.float32), mesh=mesh)
    def gather_kernel(cb_hbm, i_hbm, o_hbm):
        def body(i_vmem, o_vmem):
            pltpu.sync_copy(cb_hbm.at[i_vmem.at[0]], o_vmem)

        pltpu.emit_pipeline(
            body,
            grid=(N // window,),
            in_specs=[pl.BlockSpec((1, window), index_map=lambda i: (0, i))],
            out_specs=[pl.BlockSpec((window, C), index_map=lambda i: (i, 0))],
            core_axis_name=("core", "subcore"),
            dimension_semantics=(pltpu.PARALLEL,),
        )(i_hbm, o_hbm)

    return gather_kernel(codebook, idx_flat.reshape(1, N))


def kernel(z, codebook):
    b, c, h, w = z.shape
    z_perm = jnp.transpose(z, (0, 2, 3, 1))
    z_flat = z_perm.reshape(-1, c)
    z_sq = jnp.sum(z_flat ** 2, axis=1, keepdims=True)
    e_sq = jnp.sum(codebook ** 2, axis=1)

    cb2 = codebook * 2.0
    z3 = z.reshape(b, c, h * w)
    zsq3 = z_sq.reshape(b, 1, h * w)
    esq2 = e_sq.reshape(NCODES, 1)

    idx3, loss_sum = _dist_argmin(cb2, esq2, z3, zsq3)
    indices = idx3.reshape(b, h, w)

    z_q_rows = _sc_gather(codebook, idx3)
    z_q = z_q_rows.reshape(b, h * w, c).transpose(0, 2, 1).reshape(b, c, h, w)

    z_q_st = z + (z_q - z)
    vq_loss = (loss_sum[0, 0] / jnp.float32(z.size)) * jnp.float32(1.0 + BETA)
    return (z_q_st, vq_loss, indices)
